# Initial kernel scaffold; baseline (speedup 1.0000x reference)
import jax
import jax.numpy as jnp
from jax.experimental import pallas as pl


def kernel(x_indivi, x_event, edge_index_e2i, edge_attr_e2i, edge_index_i2e, edge_attr_i2e, nnW1, nnb1, rootW1, b1, nnW2, nnb2, rootW2, b2, nnW3, nnb3, rootW3, b3):
    raise NotImplementedError("write your pallas kernel here")


# speedup vs baseline: 4.2446x; 4.2446x over previous
import functools

import jax
import jax.numpy as jnp
from jax import lax
from jax.experimental import pallas as pl
from jax.experimental.pallas import tpu as pltpu
from jax.experimental.pallas import tpu_sc as plsc

_D = 16
_DE = 4
_NC = 2
_NS = 16
_NW = _NC * _NS
_CH = 1000
_E2 = 163840
_CH2 = 1024


def _tc_pre(x_indivi, x_event, c1, c2, r1w, r2w, b1, b2):
    n_i, n_e = x_indivi.shape[0], x_event.shape[0]

    def body(xi, xe, c1r, c2r, w1r, w2r, b1r, b2r, y1o, y2o, r1o, r2o):
        y1o[...] = jnp.dot(xe[...], c1r[...], preferred_element_type=jnp.float32)
        y2o[...] = jnp.dot(xi[...], c2r[...], preferred_element_type=jnp.float32)
        r1o[...] = jnp.dot(xi[...], w1r[...], preferred_element_type=jnp.float32) + b1r[...]
        r2o[...] = jnp.dot(xe[...], w2r[...], preferred_element_type=jnp.float32) + b2r[...]

    return pl.pallas_call(
        body,
        out_shape=[
            jax.ShapeDtypeStruct((n_e, 5 * _D), jnp.float32),
            jax.ShapeDtypeStruct((n_i, 5 * _D), jnp.float32),
            jax.ShapeDtypeStruct((n_i, _D), jnp.float32),
            jax.ShapeDtypeStruct((n_e, _D), jnp.float32),
        ],
    )(x_indivi, x_event, c1, c2, r1w, r2w, b1, b2)


def _sc_layer1(y1, s1, d1, a1, y2, s2, d2, a2):
    n_e = y1.shape[0]
    n_i = y2.shape[0]
    e = s1.shape[0]
    epw = e // _NW
    nchunk = epw // _CH
    rpt_i = n_i // _NS
    rpt_e = n_e // _NS
    mesh = plsc.VectorSubcoreMesh(core_axis_name="c", subcore_axis_name="s")

    @functools.partial(
        pl.kernel,
        out_type=[
            jax.ShapeDtypeStruct((_NW, n_i // _NS, _D), jnp.float32),
            jax.ShapeDtypeStruct((_NW, n_e // _NS, _D), jnp.float32),
        ],
        mesh=mesh,
        compiler_params=pltpu.CompilerParams(use_tc_tiling_on_sc=False, needs_layout_passes=False),
        scratch_types=[
            pltpu.VMEM((_CH,), jnp.int32),
            pltpu.VMEM((_CH,), jnp.int32),
            pltpu.VMEM((_CH * _DE,), jnp.float32),
            pltpu.VMEM((_CH, 5 * _D), jnp.float32),
            pltpu.VMEM((_CH, _D), jnp.float32),
            pltpu.VMEM_SHARED((n_i, _D), jnp.float32),
            pltpu.VMEM_SHARED((n_e, _D), jnp.float32),
            pltpu.SemaphoreType.DMA,
        ],
    )
    def k(y1h, s1h, d1h, a1h, y2h, s2h, d2h, a2h, aggi_h, agge_h,
          src_v, dst_v, attr_v, rows_v, msg_v, aggi_sh, agge_sh, sem):
        c = lax.axis_index("c")
        s = lax.axis_index("s")
        wid = c * _NS + s

        def zero_body(i, carry):
            msg_v[i] = jnp.zeros((_D,), jnp.float32)
            return carry

        lax.fori_loop(0, _CH, zero_body, 0)
        pltpu.sync_copy(msg_v.at[pl.ds(0, rpt_i)], aggi_sh.at[pl.ds(s * rpt_i, rpt_i)])
        pltpu.sync_copy(msg_v.at[pl.ds(0, rpt_e)], agge_sh.at[pl.ds(s * rpt_e, rpt_e)])
        plsc.subcore_barrier()

        def do_edges(yh, sh, dh, ah, agg_sh):
            for chunk in range(nchunk):
                base = wid * epw + chunk * _CH
                pltpu.sync_copy(sh.at[pl.ds(base, _CH)], src_v)
                pltpu.sync_copy(dh.at[pl.ds(base, _CH)], dst_v)
                pltpu.sync_copy(ah.at[pl.ds(base * _DE, _CH * _DE)], attr_v)
                pltpu.async_copy(yh.at[src_v], rows_v, sem).wait()

                def grp_body(g, carry):
                    av = attr_v[pl.ds(g * 16, 16)]
                    for t in range(4):
                        i = g * 4 + t
                        msg_v[i] = (av[4 * t] * rows_v[i, pl.ds(0, _D)]
                                    + av[4 * t + 1] * rows_v[i, pl.ds(_D, _D)]
                                    + av[4 * t + 2] * rows_v[i, pl.ds(2 * _D, _D)]
                                    + av[4 * t + 3] * rows_v[i, pl.ds(3 * _D, _D)]
                                    + rows_v[i, pl.ds(4 * _D, _D)])
                    return carry

                lax.fori_loop(0, _CH // 4, grp_body, 0)
                pltpu.sync_copy(msg_v, agg_sh.at[dst_v], add=True)

        do_edges(y1h, s1h, d1h, a1h, aggi_sh)
        do_edges(y2h, s2h, d2h, a2h, agge_sh)
        plsc.subcore_barrier()
        pltpu.sync_copy(aggi_sh.at[pl.ds(s * rpt_i, rpt_i)], aggi_h.at[wid])
        pltpu.sync_copy(agge_sh.at[pl.ds(s * rpt_e, rpt_e)], agge_h.at[wid])

    return k(y1, s1, d1, a1, y2, s2, d2, a2)


def _tc_mid(aggi, agge, r1, r2, c3, w3t, b3):
    n_i, n_e = r1.shape[0], r2.shape[0]

    def body(ai, ae, r1r, r2r, c3r, w3r, b3r, y3o, r3o):
        hi = jax.nn.sigmoid(ai[0] + ai[1] + r1r[...])
        he = jax.nn.sigmoid(ae[0] + ae[1] + r2r[...])
        y3o[...] = jnp.dot(he, c3r[...], preferred_element_type=jnp.float32)
        r3o[...] = jnp.dot(hi, w3r[...], preferred_element_type=jnp.float32) + b3r[...]

    return pl.pallas_call(
        body,
        out_shape=[
            jax.ShapeDtypeStruct((n_e, _D), jnp.float32),
            jax.ShapeDtypeStruct((n_i, 1), jnp.float32),
        ],
    )(aggi, agge, r1, r2, c3, w3t, b3)


def _sc_layer2(y3p, s3, d3, a3):
    n_i = 10000
    epw = _E2 // _NW
    nchunk = epw // _CH2
    rpt_i = n_i // _NS
    mesh = plsc.VectorSubcoreMesh(core_axis_name="c", subcore_axis_name="s")

    @functools.partial(
        pl.kernel,
        out_type=jax.ShapeDtypeStruct((_NW, n_i // _NS, _D), jnp.float32),
        mesh=mesh,
        compiler_params=pltpu.CompilerParams(use_tc_tiling_on_sc=False, needs_layout_passes=False),
        scratch_types=[
            pltpu.VMEM((_CH2,), jnp.int32),
            pltpu.VMEM((_CH2,), jnp.int32),
            pltpu.VMEM((_CH2, _DE), jnp.float32),
            pltpu.VMEM((_CH2, _D), jnp.float32),
            pltpu.VMEM((_CH2, _D), jnp.float32),
            pltpu.VMEM_SHARED((n_i, _D), jnp.float32),
            pltpu.SemaphoreType.DMA,
        ],
    )
    def k(y3h, s3h, d3h, a3h, agg_h,
          src_v, dst_v, attr_v, rows_v, msg_v, agg_sh, sem):
        c = lax.axis_index("c")
        s = lax.axis_index("s")
        wid = c * _NS + s

        def zero_body(i, carry):
            msg_v[i] = jnp.zeros((_D,), jnp.float32)
            return carry

        lax.fori_loop(0, _CH2, zero_body, 0)
        pltpu.sync_copy(msg_v.at[pl.ds(0, rpt_i)], agg_sh.at[pl.ds(s * rpt_i, rpt_i)])
        plsc.subcore_barrier()

        lanes = lax.iota(jnp.int32, _D)
        col0 = jnp.zeros((_D,), jnp.int32)

        for chunk in range(nchunk):
            base = wid * epw + chunk * _CH2
            pltpu.sync_copy(s3h.at[pl.ds(base, _CH2)], src_v)
            pltpu.sync_copy(d3h.at[pl.ds(base, _CH2)], dst_v)
            pltpu.sync_copy(a3h.at[pl.ds(base, _CH2)], attr_v)
            pltpu.async_copy(y3h.at[src_v], rows_v, sem).wait()

            def blk_body(g, carry):
                ev = g * _D + lanes
                a0 = plsc.load_gather(attr_v, [ev, col0])
                a1_ = plsc.load_gather(attr_v, [ev, col0 + 1])
                a2_ = plsc.load_gather(attr_v, [ev, col0 + 2])
                a3_ = plsc.load_gather(attr_v, [ev, col0 + 3])
                y0 = plsc.load_gather(rows_v, [ev, col0])
                y1_ = plsc.load_gather(rows_v, [ev, col0 + 1])
                y2_ = plsc.load_gather(rows_v, [ev, col0 + 2])
                y3_ = plsc.load_gather(rows_v, [ev, col0 + 3])
                y4_ = plsc.load_gather(rows_v, [ev, col0 + 4])
                m = a0 * y0 + a1_ * y1_ + a2_ * y2_ + a3_ * y3_ + y4_
                plsc.store_scatter(msg_v, [ev, col0], m)
                return carry

            lax.fori_loop(0, _CH2 // _D, blk_body, 0)
            pltpu.sync_copy(msg_v, agg_sh.at[dst_v], add=True)

        plsc.subcore_barrier()
        pltpu.sync_copy(agg_sh.at[pl.ds(s * rpt_i, rpt_i)], agg_h.at[wid])

    return k(y3p, s3, d3, a3)


def _tc_final(agg3, r3):
    n_i = r3.shape[0]

    def body(a3, r3r, outo):
        outo[...] = jax.nn.sigmoid(a3[0, :, 0:1] + a3[1, :, 0:1] + r3r[...])

    return pl.pallas_call(
        body,
        out_shape=jax.ShapeDtypeStruct((n_i, 1), jnp.float32),
    )(agg3, r3)


def kernel(x_indivi, x_event, edge_index_e2i, edge_attr_e2i, edge_index_i2e,
           edge_attr_i2e, nnW1, nnb1, rootW1, b1, nnW2, nnb2, rootW2, b2,
           nnW3, nnb3, rootW3, b3):
    n_i, n_e = x_indivi.shape[0], x_event.shape[0]
    e = edge_attr_e2i.shape[0]

    m1 = nnW1.reshape(_D, _D, _DE)
    c1 = jnp.concatenate([m1[:, :, k] for k in range(_DE)]
                         + [nnb1.reshape(_D, _D)], axis=1)
    m2 = nnW2.reshape(_D, _D, _DE)
    c2 = jnp.concatenate([m2[:, :, k] for k in range(_DE)]
                         + [nnb2.reshape(_D, _D)], axis=1)
    c3 = jnp.concatenate([nnW3, nnb3.reshape(_D, 1),
                          jnp.zeros((_D, _D - _DE - 1), jnp.float32)], axis=1)

    s1 = edge_index_e2i[0].astype(jnp.int32)
    d1 = edge_index_e2i[1].astype(jnp.int32)
    s2 = edge_index_i2e[0].astype(jnp.int32)
    d2 = edge_index_i2e[1].astype(jnp.int32)

    y1, y2, r1, r2 = _tc_pre(x_indivi, x_event, c1, c2,
                             rootW1.T, rootW2.T,
                             b1.reshape(1, _D), b2.reshape(1, _D))

    aggi, agge = _sc_layer1(y1, s1, d1, edge_attr_e2i.reshape(-1),
                            y2, s2, d2, edge_attr_i2e.reshape(-1))
    aggi = aggi.reshape(_NC, n_i, _D)
    agge = agge.reshape(_NC, n_e, _D)

    y3, r3 = _tc_mid(aggi, agge, r1, r2, c3, rootW3.T, b3.reshape(1, 1))

    pad = _E2 - e
    y3p = jnp.concatenate([y3, jnp.zeros((_D, _D), jnp.float32)], axis=0)
    s3 = jnp.concatenate([s1, jnp.full((pad,), n_e, jnp.int32)])
    d3 = jnp.concatenate([d1, jnp.zeros((pad,), jnp.int32)])
    a3 = jnp.concatenate([edge_attr_e2i, jnp.zeros((pad, _DE), jnp.float32)])

    agg3 = _sc_layer2(y3p, s3, d3, a3).reshape(_NC, n_i, _D)
    return _tc_final(agg3, r3)
